# baseline (speedup 1.0000x reference)
import jax
import jax.numpy as jnp
import numpy as np
from jax import lax
from jax.experimental import pallas as pl
from jax.experimental.pallas import tpu as pltpu
from jax.experimental.pallas import tpu_sc as plsc

VOCAB = 100000
D_MODEL = 768
SEQ_LEN = 2048
BATCH = 4

NC = 2
NS = 16
NW = NC * NS

POS_PER_W = SEQ_LEN // NW
GPOS = 8
NCHUNK = POS_PER_W // GPOS
CHUNK = GPOS * BATCH
LANES = 16
KSTEPS = D_MODEL // LANES
TOKENS = BATCH * SEQ_LEN


def _pos_encoding_np(seq_len, d_model):
    pos = np.arange(seq_len, dtype=np.float32)[:, None]
    ind = np.arange(0, d_model, 2, dtype=np.float32)
    angle = pos / (10000.0 ** (ind / d_model))
    enc = np.zeros((seq_len, d_model), dtype=np.float32)
    enc[:, 0::2] = np.sin(angle)
    enc[:, 1::2] = np.cos(angle)
    return enc


_POS_ENC = _pos_encoding_np(SEQ_LEN, D_MODEL)


NBUF = 3


def _sc_body(x_hbm, pos_hbm, table_hbm, out_hbm,
             idx_v, pos_v, rows0_v, rows1_v, rows2_v, gsem, psem, ssem):
    wid = lax.axis_index("s") * NC + lax.axis_index("c")
    p0 = wid * POS_PER_W

    pos_cp = pltpu.async_copy(
        pos_hbm.at[pl.ds(p0 * D_MODEL, POS_PER_W * D_MODEL)], pos_v, psem)
    icp = [
        pltpu.async_copy(x_hbm.at[b, pl.ds(p0, POS_PER_W)], idx_v.at[b], gsem)
        for b in range(BATCH)
    ]
    for cp in icp:
        cp.wait()

    rows = [rows0_v, rows1_v, rows2_v]
    gcp = [None] * NBUF
    scp = [None] * NBUF

    def gather(q, buf):
        return [
            pltpu.async_copy(
                table_hbm.at[idx_v.at[b, pl.ds(q * GPOS, GPOS)]],
                rows[buf].at[pl.ds(b * GPOS, GPOS)],
                gsem,
            )
            for b in range(BATCH)
        ]

    for q0 in range(NBUF - 1):
        gcp[q0] = gather(q0, q0)
    pos_cp.wait()

    for q in range(NCHUNK):
        u = q % NBUF
        un = (q + NBUF - 1) % NBUF
        if q + NBUF - 1 < NCHUNK:
            if scp[un] is not None:
                for cp in scp[un]:
                    cp.wait()
            gcp[un] = gather(q + NBUF - 1, un)
        for cp in gcp[u]:
            cp.wait()

        r = rows[u]

        def add_row(j):
            for k in range(KSTEPS):
                sl = pl.ds(k * LANES, LANES)
                v = pos_v[pl.ds((q * GPOS + j) * D_MODEL + k * LANES, LANES)]
                for b in range(BATCH):
                    plsc.addupdate(r.at[b * GPOS + j, sl], v)

        lax.fori_loop(0, GPOS, lambda j, _: (add_row(j), 0)[1], 0)

        scp[u] = [
            pltpu.async_copy(
                r.at[pl.ds(b * GPOS, GPOS)],
                out_hbm.at[b, pl.ds(p0 + q * GPOS, GPOS)],
                ssem,
            )
            for b in range(BATCH)
        ]

    for u in range(NBUF):
        if scp[u] is not None:
            for cp in scp[u]:
                cp.wait()


@jax.jit
def _embed(x, pos, table):
    mesh = plsc.VectorSubcoreMesh(
        core_axis_name="c", subcore_axis_name="s", num_cores=NC, num_subcores=NS
    )
    k = pl.kernel(
        _sc_body,
        out_type=jax.ShapeDtypeStruct((BATCH, SEQ_LEN, D_MODEL), jnp.float32),
        mesh=mesh,
        scratch_types=[
            pltpu.VMEM((BATCH, POS_PER_W), jnp.int32),
            pltpu.VMEM((POS_PER_W * D_MODEL,), jnp.float32),
            pltpu.VMEM((CHUNK, D_MODEL), jnp.float32),
            pltpu.VMEM((CHUNK, D_MODEL), jnp.float32),
            pltpu.VMEM((CHUNK, D_MODEL), jnp.float32),
            pltpu.SemaphoreType.DMA,
            pltpu.SemaphoreType.DMA,
            pltpu.SemaphoreType.DMA,
        ],
    )
    return k(x, pos, table)


def kernel(x, table):
    pos = jnp.asarray(_POS_ENC.reshape(-1))
    return _embed(x.astype(jnp.int32), pos, table)


# speedup vs baseline: 1.0512x; 1.0512x over previous
import jax
import jax.numpy as jnp
import numpy as np
from jax import lax
from jax.experimental import pallas as pl
from jax.experimental.pallas import tpu as pltpu
from jax.experimental.pallas import tpu_sc as plsc

VOCAB = 100000
D_MODEL = 768
SEQ_LEN = 2048
BATCH = 4

NC = 2
NS = 16
NW = NC * NS

POS_PER_W = SEQ_LEN // NW
GPOS = 8
NCHUNK = POS_PER_W // GPOS
CHUNK = GPOS * BATCH
LANES = 16
KSTEPS = D_MODEL // LANES
TOKENS = BATCH * SEQ_LEN


def _pos_encoding_np(seq_len, d_model):
    pos = np.arange(seq_len, dtype=np.float32)[:, None]
    ind = np.arange(0, d_model, 2, dtype=np.float32)
    angle = pos / (10000.0 ** (ind / d_model))
    enc = np.zeros((seq_len, d_model), dtype=np.float32)
    enc[:, 0::2] = np.sin(angle)
    enc[:, 1::2] = np.cos(angle)
    return enc


_POS_ENC = _pos_encoding_np(SEQ_LEN, D_MODEL)

_POS_DEV = None


def _pos_device():
    global _POS_DEV
    if _POS_DEV is None:
        _POS_DEV = jax.device_put(_POS_ENC)
    return _POS_DEV


NBUF = 3


def _sc_body(x_hbm, pos_hbm, table_hbm, out_hbm,
             idx_v, pos_v, rows0_v, rows1_v, rows2_v, gsem, psem, ssem):
    wid = lax.axis_index("s") * NC + lax.axis_index("c")
    p0 = wid * POS_PER_W

    pos_cp = pltpu.async_copy(pos_hbm.at[pl.ds(p0, POS_PER_W)], pos_v, psem)
    icp = [
        pltpu.async_copy(x_hbm.at[b, pl.ds(p0, POS_PER_W)], idx_v.at[b], gsem)
        for b in range(BATCH)
    ]
    for cp in icp:
        cp.wait()

    rows = [rows0_v, rows1_v, rows2_v]
    gcp = [None] * NBUF
    scp = [None] * NBUF

    def gather(q, buf):
        return [
            pltpu.async_copy(
                table_hbm.at[idx_v.at[b, pl.ds(q * GPOS, GPOS)]],
                rows[buf].at[pl.ds(b * GPOS, GPOS)],
                gsem,
            )
            for b in range(BATCH)
        ]

    for q0 in range(NBUF - 1):
        gcp[q0] = gather(q0, q0)
    pos_cp.wait()

    for q in range(NCHUNK):
        u = q % NBUF
        un = (q + NBUF - 1) % NBUF
        if q + NBUF - 1 < NCHUNK:
            if scp[un] is not None:
                for cp in scp[un]:
                    cp.wait()
            gcp[un] = gather(q + NBUF - 1, un)
        for cp in gcp[u]:
            cp.wait()

        r = rows[u]

        def add_row(j):
            for k in range(KSTEPS):
                sl = pl.ds(k * LANES, LANES)
                v = pos_v[q * GPOS + j, sl]
                for b in range(BATCH):
                    plsc.addupdate(r.at[b * GPOS + j, sl], v)

        lax.fori_loop(0, GPOS, lambda j, _: (add_row(j), 0)[1], 0)

        scp[u] = [
            pltpu.async_copy(
                r.at[pl.ds(b * GPOS, GPOS)],
                out_hbm.at[b, pl.ds(p0 + q * GPOS, GPOS)],
                ssem,
            )
            for b in range(BATCH)
        ]

    for u in range(NBUF):
        if scp[u] is not None:
            for cp in scp[u]:
                cp.wait()


@jax.jit
def _embed(x, pos, table):
    mesh = plsc.VectorSubcoreMesh(
        core_axis_name="c", subcore_axis_name="s", num_cores=NC, num_subcores=NS
    )
    k = pl.kernel(
        _sc_body,
        out_type=jax.ShapeDtypeStruct((BATCH, SEQ_LEN, D_MODEL), jnp.float32),
        mesh=mesh,
        scratch_types=[
            pltpu.VMEM((BATCH, POS_PER_W), jnp.int32),
            pltpu.VMEM((POS_PER_W, D_MODEL), jnp.float32),
            pltpu.VMEM((CHUNK, D_MODEL), jnp.float32),
            pltpu.VMEM((CHUNK, D_MODEL), jnp.float32),
            pltpu.VMEM((CHUNK, D_MODEL), jnp.float32),
            pltpu.SemaphoreType.DMA,
            pltpu.SemaphoreType.DMA,
            pltpu.SemaphoreType.DMA,
        ],
    )
    return k(x, pos, table)


def kernel(x, table):
    return _embed(x.astype(jnp.int32), _pos_device(), table)


# speedup vs baseline: 1.0537x; 1.0024x over previous
import jax
import jax.numpy as jnp
import numpy as np
from jax import lax
from jax.experimental import pallas as pl
from jax.experimental.pallas import tpu as pltpu
from jax.experimental.pallas import tpu_sc as plsc

VOCAB = 100000
D_MODEL = 768
SEQ_LEN = 2048
BATCH = 4

NC = 2
NS = 16
NW = NC * NS

POS_PER_W = SEQ_LEN // NW
GPOS = 8
NCHUNK = POS_PER_W // GPOS
CHUNK = GPOS * BATCH
LANES = 16
KSTEPS = D_MODEL // LANES
TOKENS = BATCH * SEQ_LEN


def _pos_encoding_np(seq_len, d_model):
    pos = np.arange(seq_len, dtype=np.float32)[:, None]
    ind = np.arange(0, d_model, 2, dtype=np.float32)
    angle = pos / (10000.0 ** (ind / d_model))
    enc = np.zeros((seq_len, d_model), dtype=np.float32)
    enc[:, 0::2] = np.sin(angle)
    enc[:, 1::2] = np.cos(angle)
    return enc


_POS_ENC = _pos_encoding_np(SEQ_LEN, D_MODEL)

_POS_DEV = None


def _pos_device():
    global _POS_DEV
    if _POS_DEV is None:
        _POS_DEV = jax.device_put(_POS_ENC)
    return _POS_DEV


NBUF = 3


def _sc_body(x_hbm, pos_hbm, table_hbm, out_hbm,
             idx_v, pos_v, rows0_v, rows1_v, rows2_v, gsem, psem, ssem):
    wid = lax.axis_index("s") * NC + lax.axis_index("c")
    p0 = wid * POS_PER_W

    pos_cp = pltpu.async_copy(pos_hbm.at[pl.ds(p0, POS_PER_W)], pos_v, psem)
    icp = [
        pltpu.async_copy(x_hbm.at[b, pl.ds(p0, POS_PER_W)], idx_v.at[b], gsem)
        for b in range(BATCH)
    ]
    for cp in icp:
        cp.wait()

    rows = [rows0_v, rows1_v, rows2_v]
    gcp = [None] * NBUF
    scp = [None] * NBUF

    def gather(q, buf):
        return [
            pltpu.async_copy(
                table_hbm.at[idx_v.at[b, pl.ds(q * GPOS, GPOS)]],
                rows[buf].at[pl.ds(b * GPOS, GPOS)],
                gsem,
            )
            for b in range(BATCH)
        ]

    for q0 in range(NBUF - 1):
        gcp[q0] = gather(q0, q0)
    pos_cp.wait()

    for q in range(NCHUNK):
        u = q % NBUF
        un = (q + NBUF - 1) % NBUF
        if q + NBUF - 1 < NCHUNK:
            if scp[un] is not None:
                for cp in scp[un]:
                    cp.wait()
            gcp[un] = gather(q + NBUF - 1, un)
        for cp in gcp[u]:
            cp.wait()

        r = rows[u]

        @plsc.parallel_loop(0, GPOS)
        def _(j):
            for k in range(KSTEPS):
                sl = pl.ds(k * LANES, LANES)
                v = pos_v[q * GPOS + j, sl]
                for b in range(BATCH):
                    plsc.addupdate(r.at[b * GPOS + j, sl], v)

        scp[u] = [
            pltpu.async_copy(
                r.at[pl.ds(b * GPOS, GPOS)],
                out_hbm.at[b, pl.ds(p0 + q * GPOS, GPOS)],
                ssem,
            )
            for b in range(BATCH)
        ]

    for u in range(NBUF):
        if scp[u] is not None:
            for cp in scp[u]:
                cp.wait()


@jax.jit
def _embed(x, pos, table):
    mesh = plsc.VectorSubcoreMesh(
        core_axis_name="c", subcore_axis_name="s", num_cores=NC, num_subcores=NS
    )
    k = pl.kernel(
        _sc_body,
        out_type=jax.ShapeDtypeStruct((BATCH, SEQ_LEN, D_MODEL), jnp.float32),
        mesh=mesh,
        scratch_types=[
            pltpu.VMEM((BATCH, POS_PER_W), jnp.int32),
            pltpu.VMEM((POS_PER_W, D_MODEL), jnp.float32),
            pltpu.VMEM((CHUNK, D_MODEL), jnp.float32),
            pltpu.VMEM((CHUNK, D_MODEL), jnp.float32),
            pltpu.VMEM((CHUNK, D_MODEL), jnp.float32),
            pltpu.SemaphoreType.DMA,
            pltpu.SemaphoreType.DMA,
            pltpu.SemaphoreType.DMA,
        ],
    )
    return k(x, pos, table)


def kernel(x, table):
    return _embed(x.astype(jnp.int32), _pos_device(), table)
